# Initial kernel scaffold; baseline (speedup 1.0000x reference)
import jax
import jax.numpy as jnp
from jax.experimental import pallas as pl


def kernel(logits, labels):
    raise NotImplementedError("write your pallas kernel here")


# speedup vs baseline: 19.1735x; 19.1735x over previous
import functools
import numpy as np
import jax
import jax.numpy as jnp
from jax import lax
from jax.experimental import pallas as pl
from jax.experimental.pallas import tpu as pltpu
from jax.experimental.pallas import tpu_sc as plsc

_N = 4194304
_N_MIN = 262144.0
_TH = float(np.float32(-np.log(np.float64(0.7))))

_ROWS, _COLS = 4096, 1024
_G1 = 32
_BLK = _ROWS // _G1

_NT = 32
_PER = _N // _NT
_CHUNK = 16384
_NCH = _PER // _CHUNK
_NB1 = 32768
_NB2 = 65536

_MESH = plsc.VectorSubcoreMesh(
    core_axis_name="c", subcore_axis_name="s", num_cores=2, num_subcores=16)


def _p1_body(x_ref, y_ref, loss_ref, sum_ref, cnt_ref):
    x = x_ref[...]
    y = y_ref[...]
    loss = jnp.maximum(x, 0.0) - x * y + jnp.log1p(jnp.exp(-jnp.abs(x)))
    loss_ref[...] = loss
    m = loss > _TH
    psum = jnp.sum(jnp.where(m, loss, 0.0))
    pcnt = jnp.sum(m.astype(jnp.float32))
    sum_ref[...] = jnp.full((1, 1, 128), psum, jnp.float32)
    cnt_ref[...] = jnp.full((1, 1, 128), pcnt, jnp.float32)


_p1 = pl.pallas_call(
    _p1_body,
    grid=(_G1,),
    in_specs=[pl.BlockSpec((_BLK, _COLS), lambda i: (i, 0)),
              pl.BlockSpec((_BLK, _COLS), lambda i: (i, 0))],
    out_specs=[pl.BlockSpec((_BLK, _COLS), lambda i: (i, 0)),
               pl.BlockSpec((1, 1, 128), lambda i: (i, 0, 0)),
               pl.BlockSpec((1, 1, 128), lambda i: (i, 0, 0))],
    out_shape=[jax.ShapeDtypeStruct((_ROWS, _COLS), jnp.float32),
               jax.ShapeDtypeStruct((_G1, 1, 128), jnp.float32),
               jax.ShapeDtypeStruct((_G1, 1, 128), jnp.float32)],
)


@functools.partial(
    pl.kernel,
    mesh=_MESH,
    compiler_params=pltpu.CompilerParams(needs_layout_passes=False),
    out_type=[jax.ShapeDtypeStruct((_NT, _NB1), jnp.int32),
              jax.ShapeDtypeStruct((_NT, _NB1), jnp.float32)],
    scratch_types=[pltpu.VMEM((_CHUNK,), jnp.float32),
                   pltpu.VMEM((_NB1,), jnp.int32),
                   pltpu.VMEM((_NB1,), jnp.float32)],
)
def _hist1(loss_hbm, cnt_out, sum_out, buf, hcnt, hsum):
    wid = lax.axis_index("s") * 2 + lax.axis_index("c")

    def zbody(i, carry):
        hcnt[pl.ds(i * 16, 16)] = jnp.zeros((16,), jnp.int32)
        hsum[pl.ds(i * 16, 16)] = jnp.zeros((16,), jnp.float32)
        return carry
    lax.fori_loop(0, _NB1 // 16, zbody, 0)

    base = wid * _PER
    ones = jnp.ones((16,), jnp.int32)

    def cbody(c, carry):
        pltpu.sync_copy(loss_hbm.at[pl.ds(base + c * _CHUNK, _CHUNK)], buf)

        def vbody(j, carry2):
            v = buf[pl.ds(j * 16, 16)]
            bits = lax.bitcast_convert_type(v, jnp.int32)
            hi = lax.shift_right_logical(bits, 16)
            plsc.addupdate_scatter(hcnt, [hi], ones)
            plsc.addupdate_scatter(hsum, [hi], v)
            return carry2
        lax.fori_loop(0, _CHUNK // 16, vbody, 0)
        return carry
    lax.fori_loop(0, _NCH, cbody, 0)

    pltpu.sync_copy(hcnt, cnt_out.at[wid])
    pltpu.sync_copy(hsum, sum_out.at[wid])


def _suffix(h):
    R = h.shape[0]
    W = h
    k = 1
    while k < 128:
        W = W + jnp.concatenate([W[:, k:], jnp.zeros((R, k), h.dtype)], axis=1)
        k *= 2
    rt = W[:, 0:1]
    T = rt
    k = 1
    while k < R:
        T = T + jnp.concatenate([T[k:, :], jnp.zeros((k, 1), h.dtype)], axis=0)
        k *= 2
    return W + (T - rt)


def _p3_body(cnt_ref, sum_ref, sumP_ref, cntP_ref, stats_ref, bvec_ref):
    h = cnt_ref[0].astype(jnp.float32)
    hs = sum_ref[0]
    for i in range(1, _NT):
        h = h + cnt_ref[i].astype(jnp.float32)
        hs = hs + sum_ref[i]
    S = _suffix(h)
    R = _NB1 // 128
    ii = lax.broadcasted_iota(jnp.int32, (R, 128), 0)
    jj = lax.broadcasted_iota(jnp.int32, (R, 128), 1)
    flat = (ii * 128 + jj).astype(jnp.float32)
    B = jnp.max(jnp.where(S >= _N_MIN, flat, -1.0))
    hB = jnp.sum(jnp.where(flat == B, h, 0.0))
    SB = jnp.sum(jnp.where(flat == B, S, 0.0))
    aboveB = SB - hB
    r = _N_MIN - aboveB
    sumAbove = jnp.sum(jnp.where(flat > B, hs, 0.0))
    sgt = jnp.sum(sumP_ref[:, 0, 0:1])
    cgt = jnp.sum(cntP_ref[:, 0, 0:1])
    lane = lax.broadcasted_iota(jnp.int32, (1, 128), 1)
    stats = (jnp.where(lane == 0, B, 0.0)
             + jnp.where(lane == 1, aboveB, 0.0)
             + jnp.where(lane == 2, r, 0.0)
             + jnp.where(lane == 3, sumAbove, 0.0)
             + jnp.where(lane == 4, cgt, 0.0)
             + jnp.where(lane == 5, sgt, 0.0))
    stats_ref[...] = stats
    bvec_ref[...] = jnp.full((8, 128), B, jnp.float32).astype(jnp.int32)


_p3 = pl.pallas_call(
    _p3_body,
    out_shape=[jax.ShapeDtypeStruct((1, 128), jnp.float32),
               jax.ShapeDtypeStruct((8, 128), jnp.int32)],
)


@functools.partial(
    pl.kernel,
    mesh=_MESH,
    compiler_params=pltpu.CompilerParams(needs_layout_passes=False),
    out_type=jax.ShapeDtypeStruct((_NT, _NB2), jnp.int32),
    scratch_types=[pltpu.VMEM((_CHUNK,), jnp.float32),
                   pltpu.VMEM((_NB2,), jnp.int32),
                   pltpu.VMEM((128,), jnp.int32)],
)
def _hist2(loss_hbm, bvec_hbm, out, buf, hist, bvbuf):
    wid = lax.axis_index("s") * 2 + lax.axis_index("c")
    pltpu.sync_copy(bvec_hbm.at[0], bvbuf)
    bv = bvbuf[pl.ds(0, 16)]

    def zbody(i, carry):
        hist[pl.ds(i * 16, 16)] = jnp.zeros((16,), jnp.int32)
        return carry
    lax.fori_loop(0, _NB2 // 16, zbody, 0)

    base = wid * _PER
    ones = jnp.ones((16,), jnp.int32)

    def cbody(c, carry):
        pltpu.sync_copy(loss_hbm.at[pl.ds(base + c * _CHUNK, _CHUNK)], buf)

        def vbody(j, carry2):
            v = buf[pl.ds(j * 16, 16)]
            bits = lax.bitcast_convert_type(v, jnp.int32)
            hi = lax.shift_right_logical(bits, 16)
            lo = jnp.bitwise_and(bits, 65535)
            plsc.addupdate_scatter(hist, [lo], ones, mask=hi == bv)
            return carry2
        lax.fori_loop(0, _CHUNK // 16, vbody, 0)
        return carry
    lax.fori_loop(0, _NCH, cbody, 0)

    pltpu.sync_copy(hist, out.at[wid])


def _p5_body(cnt2_ref, stats_ref, out_ref):
    sv = stats_ref[...]
    lane = lax.broadcasted_iota(jnp.int32, (1, 128), 1)

    def pick(k):
        return jnp.sum(jnp.where(lane == k, sv, 0.0))

    B = pick(0)
    aboveB = pick(1)
    r = pick(2)
    sumAbove = pick(3)
    cgt = pick(4)
    sgt = pick(5)

    h = cnt2_ref[0].astype(jnp.float32)
    for i in range(1, _NT):
        h = h + cnt2_ref[i].astype(jnp.float32)
    S = _suffix(h)
    R = _NB2 // 128
    ii = lax.broadcasted_iota(jnp.int32, (R, 128), 0)
    jj = lax.broadcasted_iota(jnp.int32, (R, 128), 1)
    flati = ii * 128 + jj
    flat = flati.astype(jnp.float32)
    L = jnp.max(jnp.where(S >= r, flat, -1.0))
    hL = jnp.sum(jnp.where(flat == L, h, 0.0))
    SL = jnp.sum(jnp.where(flat == L, S, 0.0))
    cnt_gt_t = aboveB + (SL - hL)
    bbits = lax.shift_left(B.astype(jnp.int32), 16)
    vals = lax.bitcast_convert_type(jnp.bitwise_or(bbits, flati), jnp.float32)
    t = jnp.sum(jnp.where(flat == L, vals, 0.0))
    sum_gt_t = sumAbove + jnp.sum(jnp.where(flat > L, h * vals, 0.0))
    else_ans = (sum_gt_t + (_N_MIN - cnt_gt_t) * t) / _N_MIN
    if_ans = sgt / jnp.maximum(cgt, 1.0)
    ans = jnp.where(cgt > _N_MIN, if_ans, else_ans)
    out_ref[...] = jnp.full((1, 128), ans, jnp.float32)


_p5 = pl.pallas_call(
    _p5_body,
    out_shape=jax.ShapeDtypeStruct((1, 128), jnp.float32),
)


@jax.jit
def kernel(logits, labels):
    x = logits.reshape(_ROWS, _COLS)
    y = labels.reshape(_ROWS, _COLS)
    loss, sumP, cntP = _p1(x, y)
    lossf = loss.reshape(_N)
    cnt1, sum1 = _hist1(lossf)
    stats, bvec = _p3(cnt1.reshape(_NT, _NB1 // 128, 128),
                      sum1.reshape(_NT, _NB1 // 128, 128), sumP, cntP)
    cnt2 = _hist2(lossf, bvec)
    out = _p5(cnt2.reshape(_NT, _NB2 // 128, 128), stats)
    return out[0, 0]


# speedup vs baseline: 26.9116x; 1.4036x over previous
import functools
import numpy as np
import jax
import jax.numpy as jnp
from jax import lax
from jax.experimental import pallas as pl
from jax.experimental.pallas import tpu as pltpu
from jax.experimental.pallas import tpu_sc as plsc

_N = 4194304
_N_MIN = 262144.0
_TH = float(np.float32(-np.log(np.float64(0.7))))

_SHAPE4 = (16, 1, 512, 512)
_G1 = 16

_NT = 32
_PER = _N // _NT
_CHUNK = 16384
_NCH = _PER // _CHUNK
_NB1 = 32768
_NB2 = 65536
_U = 8

_MESH = plsc.VectorSubcoreMesh(
    core_axis_name="c", subcore_axis_name="s", num_cores=2, num_subcores=16)


def _p1_body(x_ref, y_ref, loss_ref, sum_ref, cnt_ref):
    x = x_ref[...]
    y = y_ref[...]
    loss = jnp.maximum(x, 0.0) - x * y + jnp.log1p(jnp.exp(-jnp.abs(x)))
    loss_ref[...] = loss
    m = loss > _TH
    psum = jnp.sum(jnp.where(m, loss, 0.0))
    pcnt = jnp.sum(m.astype(jnp.float32))
    sum_ref[...] = jnp.full((1, 1, 128), psum, jnp.float32)
    cnt_ref[...] = jnp.full((1, 1, 128), pcnt, jnp.float32)


_p1 = pl.pallas_call(
    _p1_body,
    grid=(_G1,),
    in_specs=[pl.BlockSpec((1, 1, 512, 512), lambda i: (i, 0, 0, 0)),
              pl.BlockSpec((1, 1, 512, 512), lambda i: (i, 0, 0, 0))],
    out_specs=[pl.BlockSpec((1, 1, 512, 512), lambda i: (i, 0, 0, 0)),
               pl.BlockSpec((1, 1, 128), lambda i: (i, 0, 0)),
               pl.BlockSpec((1, 1, 128), lambda i: (i, 0, 0))],
    out_shape=[jax.ShapeDtypeStruct(_SHAPE4, jnp.float32),
               jax.ShapeDtypeStruct((_G1, 1, 128), jnp.float32),
               jax.ShapeDtypeStruct((_G1, 1, 128), jnp.float32)],
)


@functools.partial(
    pl.kernel,
    mesh=_MESH,
    compiler_params=pltpu.CompilerParams(needs_layout_passes=False),
    out_type=[jax.ShapeDtypeStruct((_NT, _NB1), jnp.int32),
              jax.ShapeDtypeStruct((_NT, _NB1), jnp.float32)],
    scratch_types=[pltpu.VMEM((_CHUNK,), jnp.float32),
                   pltpu.VMEM((_CHUNK,), jnp.float32),
                   pltpu.VMEM((_NB1,), jnp.int32),
                   pltpu.VMEM((_NB1,), jnp.float32),
                   pltpu.SemaphoreType.DMA,
                   pltpu.SemaphoreType.DMA],
)
def _hist1(loss_hbm, cnt_out, sum_out, buf0, buf1, hcnt, hsum, sem0, sem1):
    wid = lax.axis_index("s") * 2 + lax.axis_index("c")

    z_i = jnp.zeros((16,), jnp.int32)
    z_f = jnp.zeros((16,), jnp.float32)

    def zbody(i, carry):
        b0 = i * (16 * _U)
        for u in range(_U):
            hcnt[pl.ds(b0 + u * 16, 16)] = z_i
            hsum[pl.ds(b0 + u * 16, 16)] = z_f
        return carry
    lax.fori_loop(0, _NB1 // (16 * _U), zbody, 0)

    base = wid * _PER
    ones = jnp.ones((16,), jnp.int32)
    bufs = (buf0, buf1)
    sems = (sem0, sem1)

    def inner(buf):
        def vbody(j, carry):
            b0 = j * (16 * _U)
            for u in range(_U):
                v = buf[pl.ds(b0 + u * 16, 16)]
                bits = lax.bitcast_convert_type(v, jnp.int32)
                hi = lax.shift_right_logical(bits, 16)
                plsc.addupdate_scatter(hcnt, [hi], ones)
                plsc.addupdate_scatter(hsum, [hi], v)
            return carry
        lax.fori_loop(0, _CHUNK // (16 * _U), vbody, 0)

    cp = pltpu.async_copy(loss_hbm.at[pl.ds(base, _CHUNK)], buf0, sem0)
    for c in range(_NCH):
        nxt = None
        if c + 1 < _NCH:
            nxt = pltpu.async_copy(
                loss_hbm.at[pl.ds(base + (c + 1) * _CHUNK, _CHUNK)],
                bufs[(c + 1) % 2], sems[(c + 1) % 2])
        cp.wait()
        inner(bufs[c % 2])
        cp = nxt

    pltpu.sync_copy(hcnt, cnt_out.at[wid])
    pltpu.sync_copy(hsum, sum_out.at[wid])


def _suffix(h):
    R = h.shape[0]
    W = h
    k = 1
    while k < 128:
        W = W + jnp.concatenate([W[:, k:], jnp.zeros((R, k), h.dtype)], axis=1)
        k *= 2
    rt = W[:, 0:1]
    T = rt
    k = 1
    while k < R:
        T = T + jnp.concatenate([T[k:, :], jnp.zeros((k, 1), h.dtype)], axis=0)
        k *= 2
    return W + (T - rt)


def _p3_body(cnt_ref, sum_ref, sumP_ref, cntP_ref, stats_ref, bvec_ref):
    h = cnt_ref[0].astype(jnp.float32)
    hs = sum_ref[0]
    for i in range(1, _NT):
        h = h + cnt_ref[i].astype(jnp.float32)
        hs = hs + sum_ref[i]
    S = _suffix(h)
    R = _NB1 // 128
    ii = lax.broadcasted_iota(jnp.int32, (R, 128), 0)
    jj = lax.broadcasted_iota(jnp.int32, (R, 128), 1)
    flat = (ii * 128 + jj).astype(jnp.float32)
    B = jnp.max(jnp.where(S >= _N_MIN, flat, -1.0))
    hB = jnp.sum(jnp.where(flat == B, h, 0.0))
    SB = jnp.sum(jnp.where(flat == B, S, 0.0))
    aboveB = SB - hB
    r = _N_MIN - aboveB
    sumAbove = jnp.sum(jnp.where(flat > B, hs, 0.0))
    sgt = jnp.sum(sumP_ref[:, 0, 0:1])
    cgt = jnp.sum(cntP_ref[:, 0, 0:1])
    lane = lax.broadcasted_iota(jnp.int32, (1, 128), 1)
    stats = (jnp.where(lane == 0, B, 0.0)
             + jnp.where(lane == 1, aboveB, 0.0)
             + jnp.where(lane == 2, r, 0.0)
             + jnp.where(lane == 3, sumAbove, 0.0)
             + jnp.where(lane == 4, cgt, 0.0)
             + jnp.where(lane == 5, sgt, 0.0))
    stats_ref[...] = stats
    bvec_ref[...] = jnp.full((8, 128), B, jnp.float32).astype(jnp.int32)


_p3 = pl.pallas_call(
    _p3_body,
    out_shape=[jax.ShapeDtypeStruct((1, 128), jnp.float32),
               jax.ShapeDtypeStruct((8, 128), jnp.int32)],
)


@functools.partial(
    pl.kernel,
    mesh=_MESH,
    compiler_params=pltpu.CompilerParams(needs_layout_passes=False),
    out_type=jax.ShapeDtypeStruct((_NT, _NB2), jnp.int32),
    scratch_types=[pltpu.VMEM((_CHUNK,), jnp.float32),
                   pltpu.VMEM((_CHUNK,), jnp.float32),
                   pltpu.VMEM((_NB2,), jnp.int32),
                   pltpu.VMEM((128,), jnp.int32),
                   pltpu.SemaphoreType.DMA,
                   pltpu.SemaphoreType.DMA],
)
def _hist2(loss_hbm, bvec_hbm, out, buf0, buf1, hist, bvbuf, sem0, sem1):
    wid = lax.axis_index("s") * 2 + lax.axis_index("c")
    pltpu.sync_copy(bvec_hbm.at[0], bvbuf)
    bv = bvbuf[pl.ds(0, 16)]

    z_i = jnp.zeros((16,), jnp.int32)

    def zbody(i, carry):
        b0 = i * (16 * _U)
        for u in range(_U):
            hist[pl.ds(b0 + u * 16, 16)] = z_i
        return carry
    lax.fori_loop(0, _NB2 // (16 * _U), zbody, 0)

    base = wid * _PER
    ones = jnp.ones((16,), jnp.int32)
    bufs = (buf0, buf1)
    sems = (sem0, sem1)

    def inner(buf):
        def vbody(j, carry):
            b0 = j * (16 * _U)
            for u in range(_U):
                v = buf[pl.ds(b0 + u * 16, 16)]
                bits = lax.bitcast_convert_type(v, jnp.int32)
                hi = lax.shift_right_logical(bits, 16)
                lo = jnp.bitwise_and(bits, 65535)
                plsc.addupdate_scatter(hist, [lo], ones, mask=hi == bv)
            return carry
        lax.fori_loop(0, _CHUNK // (16 * _U), vbody, 0)

    cp = pltpu.async_copy(loss_hbm.at[pl.ds(base, _CHUNK)], buf0, sem0)
    for c in range(_NCH):
        nxt = None
        if c + 1 < _NCH:
            nxt = pltpu.async_copy(
                loss_hbm.at[pl.ds(base + (c + 1) * _CHUNK, _CHUNK)],
                bufs[(c + 1) % 2], sems[(c + 1) % 2])
        cp.wait()
        inner(bufs[c % 2])
        cp = nxt

    pltpu.sync_copy(hist, out.at[wid])


def _p5_body(cnt2_ref, stats_ref, out_ref):
    sv = stats_ref[...]
    lane = lax.broadcasted_iota(jnp.int32, (1, 128), 1)

    def pick(k):
        return jnp.sum(jnp.where(lane == k, sv, 0.0))

    B = pick(0)
    aboveB = pick(1)
    r = pick(2)
    sumAbove = pick(3)
    cgt = pick(4)
    sgt = pick(5)

    h = cnt2_ref[0].astype(jnp.float32)
    for i in range(1, _NT):
        h = h + cnt2_ref[i].astype(jnp.float32)
    S = _suffix(h)
    R = _NB2 // 128
    ii = lax.broadcasted_iota(jnp.int32, (R, 128), 0)
    jj = lax.broadcasted_iota(jnp.int32, (R, 128), 1)
    flati = ii * 128 + jj
    flat = flati.astype(jnp.float32)
    L = jnp.max(jnp.where(S >= r, flat, -1.0))
    hL = jnp.sum(jnp.where(flat == L, h, 0.0))
    SL = jnp.sum(jnp.where(flat == L, S, 0.0))
    cnt_gt_t = aboveB + (SL - hL)
    bbits = lax.shift_left(B.astype(jnp.int32), 16)
    vals = lax.bitcast_convert_type(jnp.bitwise_or(bbits, flati), jnp.float32)
    t = jnp.sum(jnp.where(flat == L, vals, 0.0))
    sum_gt_t = sumAbove + jnp.sum(jnp.where(flat > L, h * vals, 0.0))
    else_ans = (sum_gt_t + (_N_MIN - cnt_gt_t) * t) / _N_MIN
    if_ans = sgt / jnp.maximum(cgt, 1.0)
    ans = jnp.where(cgt > _N_MIN, if_ans, else_ans)
    out_ref[...] = jnp.full((1, 128), ans, jnp.float32)


_p5 = pl.pallas_call(
    _p5_body,
    out_shape=jax.ShapeDtypeStruct((1, 128), jnp.float32),
)


@jax.jit
def kernel(logits, labels):
    loss, sumP, cntP = _p1(logits, labels)
    lossf = loss.reshape(_N)
    cnt1, sum1 = _hist1(lossf)
    stats, bvec = _p3(cnt1.reshape(_NT, _NB1 // 128, 128),
                      sum1.reshape(_NT, _NB1 // 128, 128), sumP, cntP)
    cnt2 = _hist2(lossf, bvec)
    out = _p5(cnt2.reshape(_NT, _NB2 // 128, 128), stats)
    return out[0, 0]


# speedup vs baseline: 44.5889x; 1.6569x over previous
import functools
import numpy as np
import jax
import jax.numpy as jnp
from jax import lax
from jax.experimental import pallas as pl
from jax.experimental.pallas import tpu as pltpu
from jax.experimental.pallas import tpu_sc as plsc

_N = 4194304
_N_MIN = 262144.0
_TH = float(np.float32(-np.log(np.float64(0.7))))

_SHAPE4 = (16, 1, 512, 512)
_G1 = 16

_NT = 32
_PER = _N // _NT
_CHUNK = 16384
_NCH = _PER // _CHUNK
_NB1 = 32768
_NB2 = 65536
_U = 8

_MESH = plsc.VectorSubcoreMesh(
    core_axis_name="c", subcore_axis_name="s", num_cores=2, num_subcores=16)


def _p1_body(x_ref, y_ref, loss_ref, sum_ref, cnt_ref):
    x = x_ref[...]
    y = y_ref[...]
    loss = jnp.maximum(x, 0.0) - x * y + jnp.log1p(jnp.exp(-jnp.abs(x)))
    loss_ref[...] = loss
    m = loss > _TH
    psum = jnp.sum(jnp.where(m, loss, 0.0))
    pcnt = jnp.sum(m.astype(jnp.float32))
    sum_ref[...] = jnp.full((1, 1, 128), psum, jnp.float32)
    cnt_ref[...] = jnp.full((1, 1, 128), pcnt, jnp.float32)


_p1 = pl.pallas_call(
    _p1_body,
    grid=(_G1,),
    in_specs=[pl.BlockSpec((1, 1, 512, 512), lambda i: (i, 0, 0, 0)),
              pl.BlockSpec((1, 1, 512, 512), lambda i: (i, 0, 0, 0))],
    out_specs=[pl.BlockSpec((1, 1, 512, 512), lambda i: (i, 0, 0, 0)),
               pl.BlockSpec((1, 1, 128), lambda i: (i, 0, 0)),
               pl.BlockSpec((1, 1, 128), lambda i: (i, 0, 0))],
    out_shape=[jax.ShapeDtypeStruct(_SHAPE4, jnp.float32),
               jax.ShapeDtypeStruct((_G1, 1, 128), jnp.float32),
               jax.ShapeDtypeStruct((_G1, 1, 128), jnp.float32)],
)


@functools.partial(
    pl.kernel,
    mesh=_MESH,
    compiler_params=pltpu.CompilerParams(needs_layout_passes=False),
    out_type=[jax.ShapeDtypeStruct((_NT, _NB1), jnp.int32),
              jax.ShapeDtypeStruct((_NT, _NB1), jnp.float32)],
    scratch_types=[pltpu.VMEM((_CHUNK,), jnp.float32),
                   pltpu.VMEM((_CHUNK,), jnp.float32),
                   pltpu.VMEM((_NB1,), jnp.int32),
                   pltpu.VMEM((_NB1,), jnp.float32),
                   pltpu.SemaphoreType.DMA,
                   pltpu.SemaphoreType.DMA],
)
def _hist1(loss_hbm, cnt_out, sum_out, buf0, buf1, hcnt, hsum, sem0, sem1):
    wid = lax.axis_index("s") * 2 + lax.axis_index("c")

    z_i = jnp.zeros((16,), jnp.int32)
    z_f = jnp.zeros((16,), jnp.float32)

    def zbody(i, carry):
        b0 = i * (16 * _U)
        for u in range(_U):
            hcnt[pl.ds(b0 + u * 16, 16)] = z_i
            hsum[pl.ds(b0 + u * 16, 16)] = z_f
        return carry
    lax.fori_loop(0, _NB1 // (16 * _U), zbody, 0)

    base = wid * _PER
    ones = jnp.ones((16,), jnp.int32)
    bufs = (buf0, buf1)
    sems = (sem0, sem1)

    def inner(buf):
        def vbody(j, carry):
            b0 = j * (16 * _U)
            vs = [buf[pl.ds(b0 + u * 16, 16)] for u in range(_U)]
            his = [lax.shift_right_logical(
                lax.bitcast_convert_type(v, jnp.int32), 16) for v in vs]
            for u in range(_U):
                plsc.addupdate_scatter(hcnt, [his[u]], ones)
            for u in range(_U):
                plsc.addupdate_scatter(hsum, [his[u]], vs[u])
            return carry
        lax.fori_loop(0, _CHUNK // (16 * _U), vbody, 0)

    cp = pltpu.async_copy(loss_hbm.at[pl.ds(base, _CHUNK)], buf0, sem0)
    for c in range(_NCH):
        nxt = None
        if c + 1 < _NCH:
            nxt = pltpu.async_copy(
                loss_hbm.at[pl.ds(base + (c + 1) * _CHUNK, _CHUNK)],
                bufs[(c + 1) % 2], sems[(c + 1) % 2])
        cp.wait()
        inner(bufs[c % 2])
        cp = nxt

    pltpu.sync_copy(hcnt, cnt_out.at[wid])
    pltpu.sync_copy(hsum, sum_out.at[wid])


def _suffix(h):
    R = h.shape[0]
    W = h
    k = 1
    while k < 128:
        W = W + jnp.concatenate([W[:, k:], jnp.zeros((R, k), h.dtype)], axis=1)
        k *= 2
    rt = W[:, 0:1]
    T = rt
    k = 1
    while k < R:
        T = T + jnp.concatenate([T[k:, :], jnp.zeros((k, 1), h.dtype)], axis=0)
        k *= 2
    return W + (T - rt)


def _p3_body(cnt_ref, sum_ref, sumP_ref, cntP_ref, stats_ref, bvec_ref):
    h = cnt_ref[0].astype(jnp.float32)
    hs = sum_ref[0]
    for i in range(1, _NT):
        h = h + cnt_ref[i].astype(jnp.float32)
        hs = hs + sum_ref[i]
    S = _suffix(h)
    R = _NB1 // 128
    ii = lax.broadcasted_iota(jnp.int32, (R, 128), 0)
    jj = lax.broadcasted_iota(jnp.int32, (R, 128), 1)
    flat = (ii * 128 + jj).astype(jnp.float32)
    B = jnp.max(jnp.where(S >= _N_MIN, flat, -1.0))
    hB = jnp.sum(jnp.where(flat == B, h, 0.0))
    SB = jnp.sum(jnp.where(flat == B, S, 0.0))
    aboveB = SB - hB
    r = _N_MIN - aboveB
    sumAbove = jnp.sum(jnp.where(flat > B, hs, 0.0))
    sgt = jnp.sum(sumP_ref[:, 0, 0:1])
    cgt = jnp.sum(cntP_ref[:, 0, 0:1])
    lane = lax.broadcasted_iota(jnp.int32, (1, 128), 1)
    stats = (jnp.where(lane == 0, B, 0.0)
             + jnp.where(lane == 1, aboveB, 0.0)
             + jnp.where(lane == 2, r, 0.0)
             + jnp.where(lane == 3, sumAbove, 0.0)
             + jnp.where(lane == 4, cgt, 0.0)
             + jnp.where(lane == 5, sgt, 0.0))
    stats_ref[...] = stats
    bvec_ref[...] = jnp.full((8, 128), B, jnp.float32).astype(jnp.int32)


_p3 = pl.pallas_call(
    _p3_body,
    out_shape=[jax.ShapeDtypeStruct((1, 128), jnp.float32),
               jax.ShapeDtypeStruct((8, 128), jnp.int32)],
)


@functools.partial(
    pl.kernel,
    mesh=_MESH,
    compiler_params=pltpu.CompilerParams(needs_layout_passes=False),
    out_type=jax.ShapeDtypeStruct((_NT, _NB2), jnp.int32),
    scratch_types=[pltpu.VMEM((_CHUNK,), jnp.float32),
                   pltpu.VMEM((_CHUNK,), jnp.float32),
                   pltpu.VMEM((_NB2,), jnp.int32),
                   pltpu.VMEM((128,), jnp.int32),
                   pltpu.SemaphoreType.DMA,
                   pltpu.SemaphoreType.DMA],
)
def _hist2(loss_hbm, bvec_hbm, out, buf0, buf1, hist, bvbuf, sem0, sem1):
    wid = lax.axis_index("s") * 2 + lax.axis_index("c")
    pltpu.sync_copy(bvec_hbm.at[0], bvbuf)
    bv = bvbuf[pl.ds(0, 16)]

    z_i = jnp.zeros((16,), jnp.int32)

    def zbody(i, carry):
        b0 = i * (16 * _U)
        for u in range(_U):
            hist[pl.ds(b0 + u * 16, 16)] = z_i
        return carry
    lax.fori_loop(0, _NB2 // (16 * _U), zbody, 0)

    base = wid * _PER
    ones = jnp.ones((16,), jnp.int32)
    bufs = (buf0, buf1)
    sems = (sem0, sem1)

    def inner(buf):
        def vbody(j, carry):
            b0 = j * (16 * _U)
            vs = [buf[pl.ds(b0 + u * 16, 16)] for u in range(_U)]
            bs = [lax.bitcast_convert_type(v, jnp.int32) for v in vs]
            los = [jnp.bitwise_and(b, 65535) for b in bs]
            mks = [lax.shift_right_logical(b, 16) == bv for b in bs]
            for u in range(_U):
                plsc.addupdate_scatter(hist, [los[u]], ones, mask=mks[u])
            return carry
        lax.fori_loop(0, _CHUNK // (16 * _U), vbody, 0)

    cp = pltpu.async_copy(loss_hbm.at[pl.ds(base, _CHUNK)], buf0, sem0)
    for c in range(_NCH):
        nxt = None
        if c + 1 < _NCH:
            nxt = pltpu.async_copy(
                loss_hbm.at[pl.ds(base + (c + 1) * _CHUNK, _CHUNK)],
                bufs[(c + 1) % 2], sems[(c + 1) % 2])
        cp.wait()
        inner(bufs[c % 2])
        cp = nxt

    pltpu.sync_copy(hist, out.at[wid])


def _p5_body(cnt2_ref, stats_ref, out_ref):
    sv = stats_ref[...]
    lane = lax.broadcasted_iota(jnp.int32, (1, 128), 1)

    def pick(k):
        return jnp.sum(jnp.where(lane == k, sv, 0.0))

    B = pick(0)
    aboveB = pick(1)
    r = pick(2)
    sumAbove = pick(3)
    cgt = pick(4)
    sgt = pick(5)

    h = cnt2_ref[0].astype(jnp.float32)
    for i in range(1, _NT):
        h = h + cnt2_ref[i].astype(jnp.float32)
    S = _suffix(h)
    R = _NB2 // 128
    ii = lax.broadcasted_iota(jnp.int32, (R, 128), 0)
    jj = lax.broadcasted_iota(jnp.int32, (R, 128), 1)
    flati = ii * 128 + jj
    flat = flati.astype(jnp.float32)
    L = jnp.max(jnp.where(S >= r, flat, -1.0))
    hL = jnp.sum(jnp.where(flat == L, h, 0.0))
    SL = jnp.sum(jnp.where(flat == L, S, 0.0))
    cnt_gt_t = aboveB + (SL - hL)
    bbits = lax.shift_left(B.astype(jnp.int32), 16)
    vals = lax.bitcast_convert_type(jnp.bitwise_or(bbits, flati), jnp.float32)
    t = jnp.sum(jnp.where(flat == L, vals, 0.0))
    sum_gt_t = sumAbove + jnp.sum(jnp.where(flat > L, h * vals, 0.0))
    else_ans = (sum_gt_t + (_N_MIN - cnt_gt_t) * t) / _N_MIN
    if_ans = sgt / jnp.maximum(cgt, 1.0)
    ans = jnp.where(cgt > _N_MIN, if_ans, else_ans)
    out_ref[...] = jnp.full((1, 128), ans, jnp.float32)


_p5 = pl.pallas_call(
    _p5_body,
    out_shape=jax.ShapeDtypeStruct((1, 128), jnp.float32),
)


@jax.jit
def kernel(logits, labels):
    loss, sumP, cntP = _p1(logits, labels)
    lossf = loss.reshape(_N)
    cnt1, sum1 = _hist1(lossf)
    stats, bvec = _p3(cnt1.reshape(_NT, _NB1 // 128, 128),
                      sum1.reshape(_NT, _NB1 // 128, 128), sumP, cntP)
    cnt2 = _hist2(lossf, bvec)
    out = _p5(cnt2.reshape(_NT, _NB2 // 128, 128), stats)
    return out[0, 0]


# speedup vs baseline: 59.1459x; 1.3265x over previous
import functools
import numpy as np
import jax
import jax.numpy as jnp
from jax import lax
from jax.experimental import pallas as pl
from jax.experimental.pallas import tpu as pltpu
from jax.experimental.pallas import tpu_sc as plsc

_N = 4194304
_N_MIN = 262144.0
_TH = float(np.float32(-np.log(np.float64(0.7))))

_SHAPE4 = (16, 1, 512, 512)
_G1 = 16

_NT = 32
_PER = _N // _NT
_CHUNK = 16384
_NCH = _PER // _CHUNK
_NB1 = 32768
_NB2 = 65536
_U = 8

_MESH = plsc.VectorSubcoreMesh(
    core_axis_name="c", subcore_axis_name="s", num_cores=2, num_subcores=16)


def _p1_body(x_ref, y_ref, loss_ref, sum_ref, cnt_ref):
    x = x_ref[...]
    y = y_ref[...]
    loss = jnp.maximum(x, 0.0) - x * y + jnp.log1p(jnp.exp(-jnp.abs(x)))
    loss_ref[...] = loss.reshape(1, 2048, 128)
    m = loss > _TH
    psum = jnp.sum(jnp.where(m, loss, 0.0))
    pcnt = jnp.sum(m.astype(jnp.float32))
    sum_ref[...] = jnp.full((1, 1, 128), psum, jnp.float32)
    cnt_ref[...] = jnp.full((1, 1, 128), pcnt, jnp.float32)


_p1 = pl.pallas_call(
    _p1_body,
    grid=(_G1,),
    in_specs=[pl.BlockSpec((1, 1, 512, 512), lambda i: (i, 0, 0, 0)),
              pl.BlockSpec((1, 1, 512, 512), lambda i: (i, 0, 0, 0))],
    out_specs=[pl.BlockSpec((1, 2048, 128), lambda i: (i, 0, 0)),
               pl.BlockSpec((1, 1, 128), lambda i: (i, 0, 0)),
               pl.BlockSpec((1, 1, 128), lambda i: (i, 0, 0))],
    out_shape=[jax.ShapeDtypeStruct((16, 2048, 128), jnp.float32),
               jax.ShapeDtypeStruct((_G1, 1, 128), jnp.float32),
               jax.ShapeDtypeStruct((_G1, 1, 128), jnp.float32)],
)


@functools.partial(
    pl.kernel,
    mesh=_MESH,
    compiler_params=pltpu.CompilerParams(needs_layout_passes=False),
    out_type=[jax.ShapeDtypeStruct((_NT, _NB1 // 128, 128), jnp.int32),
              jax.ShapeDtypeStruct((_NT, _NB1 // 128, 128), jnp.float32)],
    scratch_types=[pltpu.VMEM((_CHUNK,), jnp.float32),
                   pltpu.VMEM((_CHUNK,), jnp.float32),
                   pltpu.VMEM((_NB1 // 128, 128), jnp.int32),
                   pltpu.VMEM((_NB1 // 128, 128), jnp.float32),
                   pltpu.SemaphoreType.DMA,
                   pltpu.SemaphoreType.DMA],
)
def _hist1(loss_hbm, cnt_out, sum_out, buf0, buf1, hcnt, hsum, sem0, sem1):
    wid = lax.axis_index("s") * 2 + lax.axis_index("c")

    z_i = jnp.zeros((16,), jnp.int32)
    z_f = jnp.zeros((16,), jnp.float32)

    def zbody(i, carry):
        for u in range(_U):
            c0 = u * 16
            hcnt[i, pl.ds(c0, 16)] = z_i
            hsum[i, pl.ds(c0, 16)] = z_f
        return carry
    lax.fori_loop(0, _NB1 // 128, zbody, 0)

    base = wid * _PER
    ones = jnp.ones((16,), jnp.int32)
    bufs = (buf0, buf1)
    sems = (sem0, sem1)

    def inner(buf):
        def vbody(j, carry):
            b0 = j * (16 * _U)
            vs = [buf[pl.ds(b0 + u * 16, 16)] for u in range(_U)]
            his = [lax.shift_right_logical(
                lax.bitcast_convert_type(v, jnp.int32), 16) for v in vs]
            rows = [lax.shift_right_logical(h, 7) for h in his]
            cols = [jnp.bitwise_and(h, 127) for h in his]
            for u in range(_U):
                plsc.addupdate_scatter(hcnt, [rows[u], cols[u]], ones)
            for u in range(_U):
                plsc.addupdate_scatter(hsum, [rows[u], cols[u]], vs[u])
            return carry
        lax.fori_loop(0, _CHUNK // (16 * _U), vbody, 0)

    cp = pltpu.async_copy(loss_hbm.at[pl.ds(base, _CHUNK)], buf0, sem0)
    for c in range(_NCH):
        nxt = None
        if c + 1 < _NCH:
            nxt = pltpu.async_copy(
                loss_hbm.at[pl.ds(base + (c + 1) * _CHUNK, _CHUNK)],
                bufs[(c + 1) % 2], sems[(c + 1) % 2])
        cp.wait()
        inner(bufs[c % 2])
        cp = nxt

    pltpu.sync_copy(hcnt, cnt_out.at[wid])
    pltpu.sync_copy(hsum, sum_out.at[wid])


def _suffix(h):
    R = h.shape[0]
    W = h
    k = 1
    while k < 128:
        W = W + jnp.concatenate([W[:, k:], jnp.zeros((R, k), h.dtype)], axis=1)
        k *= 2
    rt = W[:, 0:1]
    T = rt
    k = 1
    while k < R:
        T = T + jnp.concatenate([T[k:, :], jnp.zeros((k, 1), h.dtype)], axis=0)
        k *= 2
    return W + (T - rt)


def _p3_body(cnt_ref, sum_ref, sumP_ref, cntP_ref, stats_ref, bvec_ref):
    h = cnt_ref[0].astype(jnp.float32)
    hs = sum_ref[0]
    for i in range(1, _NT):
        h = h + cnt_ref[i].astype(jnp.float32)
        hs = hs + sum_ref[i]
    S = _suffix(h)
    R = _NB1 // 128
    ii = lax.broadcasted_iota(jnp.int32, (R, 128), 0)
    jj = lax.broadcasted_iota(jnp.int32, (R, 128), 1)
    flat = (ii * 128 + jj).astype(jnp.float32)
    B = jnp.max(jnp.where(S >= _N_MIN, flat, -1.0))
    hB = jnp.sum(jnp.where(flat == B, h, 0.0))
    SB = jnp.sum(jnp.where(flat == B, S, 0.0))
    aboveB = SB - hB
    r = _N_MIN - aboveB
    sumAbove = jnp.sum(jnp.where(flat > B, hs, 0.0))
    sgt = jnp.sum(sumP_ref[:, 0, 0:1])
    cgt = jnp.sum(cntP_ref[:, 0, 0:1])
    lane = lax.broadcasted_iota(jnp.int32, (1, 128), 1)
    stats = (jnp.where(lane == 0, B, 0.0)
             + jnp.where(lane == 1, aboveB, 0.0)
             + jnp.where(lane == 2, r, 0.0)
             + jnp.where(lane == 3, sumAbove, 0.0)
             + jnp.where(lane == 4, cgt, 0.0)
             + jnp.where(lane == 5, sgt, 0.0))
    stats_ref[...] = stats
    bvec_ref[...] = jnp.full((8, 128), B, jnp.float32).astype(jnp.int32)


_p3 = pl.pallas_call(
    _p3_body,
    out_shape=[jax.ShapeDtypeStruct((1, 128), jnp.float32),
               jax.ShapeDtypeStruct((8, 128), jnp.int32)],
)


@functools.partial(
    pl.kernel,
    mesh=_MESH,
    compiler_params=pltpu.CompilerParams(needs_layout_passes=False),
    out_type=jax.ShapeDtypeStruct((_NT, _NB2 // 128, 128), jnp.int32),
    scratch_types=[pltpu.VMEM((_CHUNK,), jnp.float32),
                   pltpu.VMEM((_CHUNK,), jnp.float32),
                   pltpu.VMEM((_NB2 // 128, 128), jnp.int32),
                   pltpu.VMEM((128,), jnp.int32),
                   pltpu.SemaphoreType.DMA,
                   pltpu.SemaphoreType.DMA],
)
def _hist2(loss_hbm, bvec_hbm, out, buf0, buf1, hist, bvbuf, sem0, sem1):
    wid = lax.axis_index("s") * 2 + lax.axis_index("c")
    pltpu.sync_copy(bvec_hbm.at[0], bvbuf)
    bv = bvbuf[pl.ds(0, 16)]

    z_i = jnp.zeros((16,), jnp.int32)

    def zbody(i, carry):
        for u in range(_U):
            hist[i, pl.ds(u * 16, 16)] = z_i
        return carry
    lax.fori_loop(0, _NB2 // 128, zbody, 0)

    base = wid * _PER
    ones = jnp.ones((16,), jnp.int32)
    bufs = (buf0, buf1)
    sems = (sem0, sem1)

    def inner(buf):
        def vbody(j, carry):
            b0 = j * (16 * _U)
            vs = [buf[pl.ds(b0 + u * 16, 16)] for u in range(_U)]
            bs = [lax.bitcast_convert_type(v, jnp.int32) for v in vs]
            rows = [jnp.bitwise_and(lax.shift_right_logical(b, 7), 511)
                    for b in bs]
            cols = [jnp.bitwise_and(b, 127) for b in bs]
            mks = [lax.shift_right_logical(b, 16) == bv for b in bs]
            for u in range(_U):
                plsc.addupdate_scatter(hist, [rows[u], cols[u]], ones,
                                       mask=mks[u])
            return carry
        lax.fori_loop(0, _CHUNK // (16 * _U), vbody, 0)

    cp = pltpu.async_copy(loss_hbm.at[pl.ds(base, _CHUNK)], buf0, sem0)
    for c in range(_NCH):
        nxt = None
        if c + 1 < _NCH:
            nxt = pltpu.async_copy(
                loss_hbm.at[pl.ds(base + (c + 1) * _CHUNK, _CHUNK)],
                bufs[(c + 1) % 2], sems[(c + 1) % 2])
        cp.wait()
        inner(bufs[c % 2])
        cp = nxt

    pltpu.sync_copy(hist, out.at[wid])


def _p5_body(cnt2_ref, stats_ref, out_ref):
    sv = stats_ref[...]
    lane = lax.broadcasted_iota(jnp.int32, (1, 128), 1)

    def pick(k):
        return jnp.sum(jnp.where(lane == k, sv, 0.0))

    B = pick(0)
    aboveB = pick(1)
    r = pick(2)
    sumAbove = pick(3)
    cgt = pick(4)
    sgt = pick(5)

    h = cnt2_ref[0].astype(jnp.float32)
    for i in range(1, _NT):
        h = h + cnt2_ref[i].astype(jnp.float32)
    S = _suffix(h)
    R = _NB2 // 128
    ii = lax.broadcasted_iota(jnp.int32, (R, 128), 0)
    jj = lax.broadcasted_iota(jnp.int32, (R, 128), 1)
    flati = ii * 128 + jj
    flat = flati.astype(jnp.float32)
    L = jnp.max(jnp.where(S >= r, flat, -1.0))
    hL = jnp.sum(jnp.where(flat == L, h, 0.0))
    SL = jnp.sum(jnp.where(flat == L, S, 0.0))
    cnt_gt_t = aboveB + (SL - hL)
    bbits = lax.shift_left(B.astype(jnp.int32), 16)
    vals = lax.bitcast_convert_type(jnp.bitwise_or(bbits, flati), jnp.float32)
    t = jnp.sum(jnp.where(flat == L, vals, 0.0))
    sum_gt_t = sumAbove + jnp.sum(jnp.where(flat > L, h * vals, 0.0))
    else_ans = (sum_gt_t + (_N_MIN - cnt_gt_t) * t) / _N_MIN
    if_ans = sgt / jnp.maximum(cgt, 1.0)
    ans = jnp.where(cgt > _N_MIN, if_ans, else_ans)
    out_ref[...] = jnp.full((1, 128), ans, jnp.float32)


_p5 = pl.pallas_call(
    _p5_body,
    out_shape=jax.ShapeDtypeStruct((1, 128), jnp.float32),
)


@jax.jit
def kernel(logits, labels):
    loss, sumP, cntP = _p1(logits, labels)
    lossf = loss.reshape(_N)
    cnt1, sum1 = _hist1(lossf)
    stats, bvec = _p3(cnt1, sum1, sumP, cntP)
    cnt2 = _hist2(lossf, bvec)
    out = _p5(cnt2, stats)
    return out[0, 0]


# speedup vs baseline: 67.1140x; 1.1347x over previous
import functools
import numpy as np
import jax
import jax.numpy as jnp
from jax import lax
from jax.experimental import pallas as pl
from jax.experimental.pallas import tpu as pltpu
from jax.experimental.pallas import tpu_sc as plsc

_N = 4194304
_N_MIN = 262144.0
_TH = float(np.float32(-np.log(np.float64(0.7))))

_SHAPE4 = (16, 1, 512, 512)
_G1 = 16

_NT = 32
_PER = _N // _NT
_CHUNK = 16384
_NCH = _PER // _CHUNK
_NB1 = 32768
_NB2 = 65536
_U = 8

_MESH = plsc.VectorSubcoreMesh(
    core_axis_name="c", subcore_axis_name="s", num_cores=2, num_subcores=16)


def _p1_body(x_ref, y_ref, loss_ref, sum_ref, cnt_ref):
    x = x_ref[...]
    y = y_ref[...]
    loss = jnp.maximum(x, 0.0) - x * y + jnp.log1p(jnp.exp(-jnp.abs(x)))
    loss_ref[...] = loss.reshape(1, 2048, 128)
    m = loss > _TH
    psum = jnp.sum(jnp.where(m, loss, 0.0))
    pcnt = jnp.sum(m.astype(jnp.float32))
    sum_ref[...] = jnp.full((1, 1, 128), psum, jnp.float32)
    cnt_ref[...] = jnp.full((1, 1, 128), pcnt, jnp.float32)


_p1 = pl.pallas_call(
    _p1_body,
    grid=(_G1,),
    in_specs=[pl.BlockSpec((1, 1, 512, 512), lambda i: (i, 0, 0, 0)),
              pl.BlockSpec((1, 1, 512, 512), lambda i: (i, 0, 0, 0))],
    out_specs=[pl.BlockSpec((1, 2048, 128), lambda i: (i, 0, 0)),
               pl.BlockSpec((1, 1, 128), lambda i: (i, 0, 0)),
               pl.BlockSpec((1, 1, 128), lambda i: (i, 0, 0))],
    out_shape=[jax.ShapeDtypeStruct((16, 2048, 128), jnp.float32),
               jax.ShapeDtypeStruct((_G1, 1, 128), jnp.float32),
               jax.ShapeDtypeStruct((_G1, 1, 128), jnp.float32)],
)


@functools.partial(
    pl.kernel,
    mesh=_MESH,
    compiler_params=pltpu.CompilerParams(needs_layout_passes=False),
    out_type=jax.ShapeDtypeStruct((_NT, _NB1 // 128, 128), jnp.int32),
    scratch_types=[pltpu.VMEM((_CHUNK,), jnp.float32),
                   pltpu.VMEM((_CHUNK,), jnp.float32),
                   pltpu.VMEM((_NB1 // 128, 128), jnp.int32),
                   pltpu.SemaphoreType.DMA,
                   pltpu.SemaphoreType.DMA],
)
def _hist1(loss_hbm, cnt_out, buf0, buf1, hcnt, sem0, sem1):
    wid = lax.axis_index("s") * 2 + lax.axis_index("c")

    z_i = jnp.zeros((16,), jnp.int32)

    def zbody(i, carry):
        for u in range(_U):
            hcnt[i, pl.ds(u * 16, 16)] = z_i
        return carry
    lax.fori_loop(0, _NB1 // 128, zbody, 0)

    base = wid * _PER
    ones = jnp.ones((16,), jnp.int32)
    bufs = (buf0, buf1)
    sems = (sem0, sem1)

    def inner(buf):
        def vbody(j, carry):
            b0 = j * (16 * _U)
            vs = [buf[pl.ds(b0 + u * 16, 16)] for u in range(_U)]
            his = [lax.shift_right_logical(
                lax.bitcast_convert_type(v, jnp.int32), 16) for v in vs]
            rows = [lax.shift_right_logical(h, 7) for h in his]
            cols = [jnp.bitwise_and(h, 127) for h in his]
            for u in range(_U):
                plsc.addupdate_scatter(hcnt, [rows[u], cols[u]], ones)
            return carry
        lax.fori_loop(0, _CHUNK // (16 * _U), vbody, 0)

    cp = pltpu.async_copy(loss_hbm.at[pl.ds(base, _CHUNK)], buf0, sem0)
    for c in range(_NCH):
        nxt = None
        if c + 1 < _NCH:
            nxt = pltpu.async_copy(
                loss_hbm.at[pl.ds(base + (c + 1) * _CHUNK, _CHUNK)],
                bufs[(c + 1) % 2], sems[(c + 1) % 2])
        cp.wait()
        inner(bufs[c % 2])
        cp = nxt

    pltpu.sync_copy(hcnt, cnt_out.at[wid])


def _suffix(h):
    R = h.shape[0]
    W = h
    k = 1
    while k < 128:
        W = W + jnp.concatenate([W[:, k:], jnp.zeros((R, k), h.dtype)], axis=1)
        k *= 2
    rt = W[:, 0:1]
    T = rt
    k = 1
    while k < R:
        T = T + jnp.concatenate([T[k:, :], jnp.zeros((k, 1), h.dtype)], axis=0)
        k *= 2
    return W + (T - rt)


def _p3_body(cnt_ref, sumP_ref, cntP_ref, stats_ref, bvec_ref):
    h = cnt_ref[0].astype(jnp.float32)
    for i in range(1, _NT):
        h = h + cnt_ref[i].astype(jnp.float32)
    S = _suffix(h)
    R = _NB1 // 128
    ii = lax.broadcasted_iota(jnp.int32, (R, 128), 0)
    jj = lax.broadcasted_iota(jnp.int32, (R, 128), 1)
    flat = (ii * 128 + jj).astype(jnp.float32)
    B = jnp.max(jnp.where(S >= _N_MIN, flat, -1.0))
    hB = jnp.sum(jnp.where(flat == B, h, 0.0))
    SB = jnp.sum(jnp.where(flat == B, S, 0.0))
    aboveB = SB - hB
    r = _N_MIN - aboveB
    sgt = jnp.sum(sumP_ref[:, 0, 0:1])
    cgt = jnp.sum(cntP_ref[:, 0, 0:1])
    lane = lax.broadcasted_iota(jnp.int32, (1, 128), 1)
    stats = (jnp.where(lane == 0, B, 0.0)
             + jnp.where(lane == 1, aboveB, 0.0)
             + jnp.where(lane == 2, r, 0.0)
             + jnp.where(lane == 4, cgt, 0.0)
             + jnp.where(lane == 5, sgt, 0.0))
    stats_ref[...] = stats
    bvec_ref[...] = jnp.full((8, 128), B, jnp.float32).astype(jnp.int32)


_p3 = pl.pallas_call(
    _p3_body,
    out_shape=[jax.ShapeDtypeStruct((1, 128), jnp.float32),
               jax.ShapeDtypeStruct((8, 128), jnp.int32)],
)


@functools.partial(
    pl.kernel,
    mesh=_MESH,
    compiler_params=pltpu.CompilerParams(needs_layout_passes=False),
    out_type=[jax.ShapeDtypeStruct((_NT, _NB2 // 128, 128), jnp.int32),
              jax.ShapeDtypeStruct((_NT, 128), jnp.float32)],
    scratch_types=[pltpu.VMEM((_CHUNK,), jnp.float32),
                   pltpu.VMEM((_CHUNK,), jnp.float32),
                   pltpu.VMEM((_NB2 // 128, 128), jnp.int32),
                   pltpu.VMEM((128,), jnp.int32),
                   pltpu.VMEM((128,), jnp.float32),
                   pltpu.SemaphoreType.DMA,
                   pltpu.SemaphoreType.DMA],
)
def _hist2(loss_hbm, bvec_hbm, out, sa_out, buf0, buf1, hist, bvbuf, abuf,
           sem0, sem1):
    wid = lax.axis_index("s") * 2 + lax.axis_index("c")
    pltpu.sync_copy(bvec_hbm.at[0], bvbuf)
    bv = bvbuf[pl.ds(0, 16)]

    z_i = jnp.zeros((16,), jnp.int32)

    def zbody(i, carry):
        for u in range(_U):
            hist[i, pl.ds(u * 16, 16)] = z_i
        return carry
    lax.fori_loop(0, _NB2 // 128, zbody, 0)

    base = wid * _PER
    ones = jnp.ones((16,), jnp.int32)
    bufs = (buf0, buf1)
    sems = (sem0, sem1)

    def inner(buf, acc0):
        def vbody(j, acc):
            b0 = j * (16 * _U)
            vs = [buf[pl.ds(b0 + u * 16, 16)] for u in range(_U)]
            bs = [lax.bitcast_convert_type(v, jnp.int32) for v in vs]
            hs = [lax.shift_right_logical(b, 16) for b in bs]
            rows = [jnp.bitwise_and(lax.shift_right_logical(b, 7), 511)
                    for b in bs]
            cols = [jnp.bitwise_and(b, 127) for b in bs]
            mks = [h == bv for h in hs]
            for u in range(_U):
                plsc.addupdate_scatter(hist, [rows[u], cols[u]], ones,
                                       mask=mks[u])
            for u in range(_U):
                acc = acc + jnp.where(hs[u] > bv, vs[u], 0.0)
            return acc
        return lax.fori_loop(0, _CHUNK // (16 * _U), vbody, acc0)

    acc = jnp.zeros((16,), jnp.float32)
    cp = pltpu.async_copy(loss_hbm.at[pl.ds(base, _CHUNK)], buf0, sem0)
    for c in range(_NCH):
        nxt = None
        if c + 1 < _NCH:
            nxt = pltpu.async_copy(
                loss_hbm.at[pl.ds(base + (c + 1) * _CHUNK, _CHUNK)],
                bufs[(c + 1) % 2], sems[(c + 1) % 2])
        cp.wait()
        acc = inner(bufs[c % 2], acc)
        cp = nxt

    abuf[pl.ds(0, 16)] = acc
    z_f = jnp.zeros((16,), jnp.float32)
    for u in range(1, _U):
        abuf[pl.ds(u * 16, 16)] = z_f
    pltpu.sync_copy(hist, out.at[wid])
    pltpu.sync_copy(abuf, sa_out.at[wid])


def _p5_body(cnt2_ref, sa_ref, stats_ref, out_ref):
    sv = stats_ref[...]
    lane = lax.broadcasted_iota(jnp.int32, (1, 128), 1)

    def pick(k):
        return jnp.sum(jnp.where(lane == k, sv, 0.0))

    B = pick(0)
    aboveB = pick(1)
    r = pick(2)
    cgt = pick(4)
    sgt = pick(5)
    sumAbove = jnp.sum(sa_ref[...])

    h = cnt2_ref[0].astype(jnp.float32)
    for i in range(1, _NT):
        h = h + cnt2_ref[i].astype(jnp.float32)
    S = _suffix(h)
    R = _NB2 // 128
    ii = lax.broadcasted_iota(jnp.int32, (R, 128), 0)
    jj = lax.broadcasted_iota(jnp.int32, (R, 128), 1)
    flati = ii * 128 + jj
    flat = flati.astype(jnp.float32)
    L = jnp.max(jnp.where(S >= r, flat, -1.0))
    hL = jnp.sum(jnp.where(flat == L, h, 0.0))
    SL = jnp.sum(jnp.where(flat == L, S, 0.0))
    cnt_gt_t = aboveB + (SL - hL)
    bbits = lax.shift_left(B.astype(jnp.int32), 16)
    vals = lax.bitcast_convert_type(jnp.bitwise_or(bbits, flati), jnp.float32)
    t = jnp.sum(jnp.where(flat == L, vals, 0.0))
    sum_gt_t = sumAbove + jnp.sum(jnp.where(flat > L, h * vals, 0.0))
    else_ans = (sum_gt_t + (_N_MIN - cnt_gt_t) * t) / _N_MIN
    if_ans = sgt / jnp.maximum(cgt, 1.0)
    ans = jnp.where(cgt > _N_MIN, if_ans, else_ans)
    out_ref[...] = jnp.full((1, 128), ans, jnp.float32)


_p5 = pl.pallas_call(
    _p5_body,
    out_shape=jax.ShapeDtypeStruct((1, 128), jnp.float32),
)


@jax.jit
def kernel(logits, labels):
    loss, sumP, cntP = _p1(logits, labels)
    lossf = loss.reshape(_N)
    cnt1 = _hist1(lossf)
    stats, bvec = _p3(cnt1, sumP, cntP)
    cnt2, sa = _hist2(lossf, bvec)
    out = _p5(cnt2, sa, stats)
    return out[0, 0]


# speedup vs baseline: 68.6745x; 1.0233x over previous
import functools
import numpy as np
import jax
import jax.numpy as jnp
from jax import lax
from jax.experimental import pallas as pl
from jax.experimental.pallas import tpu as pltpu
from jax.experimental.pallas import tpu_sc as plsc

_N = 4194304
_N_MIN = 262144.0
_TH = float(np.float32(-np.log(np.float64(0.7))))

_SHAPE4 = (16, 1, 512, 512)
_G1 = 16

_NT = 32
_PER = _N // _NT
_CHUNK = 16384
_NCH = _PER // _CHUNK
_NB1 = 32768
_NB2 = 65536
_U = 8

_MESH = plsc.VectorSubcoreMesh(
    core_axis_name="c", subcore_axis_name="s", num_cores=2, num_subcores=16)


def _p1_body(x_ref, y_ref, loss_ref, sum_ref, cnt_ref):
    x = x_ref[...]
    y = y_ref[...]
    loss = jnp.maximum(x, 0.0) - x * y + jnp.log(1.0 + jnp.exp(-jnp.abs(x)))
    loss_ref[...] = loss.reshape(4, 1, 512, 128)
    m = loss > _TH
    pcnt = jnp.sum(m.astype(jnp.float32))
    psum = jnp.sum(jnp.maximum(loss - _TH, 0.0)) + _TH * pcnt
    sum_ref[...] = jnp.full((1, 1, 128), psum, jnp.float32)
    cnt_ref[...] = jnp.full((1, 1, 128), pcnt, jnp.float32)


_p1 = pl.pallas_call(
    _p1_body,
    grid=(4, 4),
    in_specs=[pl.BlockSpec((4, 1, 512, 128), lambda i, j: (i, 0, 0, j)),
              pl.BlockSpec((4, 1, 512, 128), lambda i, j: (i, 0, 0, j))],
    out_specs=[pl.BlockSpec((4, 1, 512, 128), lambda i, j: (i, j, 0, 0)),
               pl.BlockSpec((1, 1, 128), lambda i, j: (i * 4 + j, 0, 0)),
               pl.BlockSpec((1, 1, 128), lambda i, j: (i * 4 + j, 0, 0))],
    out_shape=[jax.ShapeDtypeStruct((16, 4, 512, 128), jnp.float32),
               jax.ShapeDtypeStruct((16, 1, 128), jnp.float32),
               jax.ShapeDtypeStruct((16, 1, 128), jnp.float32)],
)


@functools.partial(
    pl.kernel,
    mesh=_MESH,
    compiler_params=pltpu.CompilerParams(needs_layout_passes=False),
    out_type=jax.ShapeDtypeStruct((_NT, _NB1 // 128, 128), jnp.int32),
    scratch_types=[pltpu.VMEM((_CHUNK,), jnp.float32),
                   pltpu.VMEM((_CHUNK,), jnp.float32),
                   pltpu.VMEM((_NB1 // 128, 128), jnp.int32),
                   pltpu.SemaphoreType.DMA,
                   pltpu.SemaphoreType.DMA],
)
def _hist1(loss_hbm, cnt_out, buf0, buf1, hcnt, sem0, sem1):
    wid = lax.axis_index("s") * 2 + lax.axis_index("c")

    z_i = jnp.zeros((16,), jnp.int32)

    def zbody(i, carry):
        for u in range(_U):
            hcnt[i, pl.ds(u * 16, 16)] = z_i
        return carry
    lax.fori_loop(0, _NB1 // 128, zbody, 0)

    base = wid * _PER
    ones = jnp.ones((16,), jnp.int32)
    bufs = (buf0, buf1)
    sems = (sem0, sem1)

    def inner(buf):
        def vbody(j, carry):
            b0 = j * (16 * _U)
            vs = [buf[pl.ds(b0 + u * 16, 16)] for u in range(_U)]
            his = [lax.shift_right_logical(
                lax.bitcast_convert_type(v, jnp.int32), 16) for v in vs]
            rows = [lax.shift_right_logical(h, 7) for h in his]
            cols = [jnp.bitwise_and(h, 127) for h in his]
            for u in range(_U):
                plsc.addupdate_scatter(hcnt, [rows[u], cols[u]], ones)
            return carry
        lax.fori_loop(0, _CHUNK // (16 * _U), vbody, 0)

    cp = pltpu.async_copy(loss_hbm.at[pl.ds(base, _CHUNK)], buf0, sem0)
    for c in range(_NCH):
        nxt = None
        if c + 1 < _NCH:
            nxt = pltpu.async_copy(
                loss_hbm.at[pl.ds(base + (c + 1) * _CHUNK, _CHUNK)],
                bufs[(c + 1) % 2], sems[(c + 1) % 2])
        cp.wait()
        inner(bufs[c % 2])
        cp = nxt

    pltpu.sync_copy(hcnt, cnt_out.at[wid])


def _suffix(h):
    R = h.shape[0]
    W = h
    k = 1
    while k < 128:
        W = W + jnp.concatenate([W[:, k:], jnp.zeros((R, k), h.dtype)], axis=1)
        k *= 2
    rt = W[:, 0:1]
    T = rt
    k = 1
    while k < R:
        T = T + jnp.concatenate([T[k:, :], jnp.zeros((k, 1), h.dtype)], axis=0)
        k *= 2
    return W + (T - rt)


def _p3_body(cnt_ref, sumP_ref, cntP_ref, stats_ref, bvec_ref):
    h = cnt_ref[0].astype(jnp.float32)
    for i in range(1, _NT):
        h = h + cnt_ref[i].astype(jnp.float32)
    S = _suffix(h)
    R = _NB1 // 128
    ii = lax.broadcasted_iota(jnp.int32, (R, 128), 0)
    jj = lax.broadcasted_iota(jnp.int32, (R, 128), 1)
    flat = (ii * 128 + jj).astype(jnp.float32)
    B = jnp.max(jnp.where(S >= _N_MIN, flat, -1.0))
    hB = jnp.sum(jnp.where(flat == B, h, 0.0))
    SB = jnp.sum(jnp.where(flat == B, S, 0.0))
    aboveB = SB - hB
    r = _N_MIN - aboveB
    sgt = jnp.sum(sumP_ref[:, 0, 0:1])
    cgt = jnp.sum(cntP_ref[:, 0, 0:1])
    lane = lax.broadcasted_iota(jnp.int32, (1, 128), 1)
    stats = (jnp.where(lane == 0, B, 0.0)
             + jnp.where(lane == 1, aboveB, 0.0)
             + jnp.where(lane == 2, r, 0.0)
             + jnp.where(lane == 4, cgt, 0.0)
             + jnp.where(lane == 5, sgt, 0.0))
    stats_ref[...] = stats
    bvec_ref[...] = jnp.full((8, 128), B, jnp.float32).astype(jnp.int32)


_p3 = pl.pallas_call(
    _p3_body,
    out_shape=[jax.ShapeDtypeStruct((1, 128), jnp.float32),
               jax.ShapeDtypeStruct((8, 128), jnp.int32)],
)


@functools.partial(
    pl.kernel,
    mesh=_MESH,
    compiler_params=pltpu.CompilerParams(needs_layout_passes=False),
    out_type=[jax.ShapeDtypeStruct((_NT, _NB2 // 128, 128), jnp.int32),
              jax.ShapeDtypeStruct((_NT, 128), jnp.float32)],
    scratch_types=[pltpu.VMEM((_CHUNK,), jnp.float32),
                   pltpu.VMEM((_CHUNK,), jnp.float32),
                   pltpu.VMEM((_NB2 // 128, 128), jnp.int32),
                   pltpu.VMEM((128,), jnp.int32),
                   pltpu.VMEM((128,), jnp.float32),
                   pltpu.SemaphoreType.DMA,
                   pltpu.SemaphoreType.DMA],
)
def _hist2(loss_hbm, bvec_hbm, out, sa_out, buf0, buf1, hist, bvbuf, abuf,
           sem0, sem1):
    wid = lax.axis_index("s") * 2 + lax.axis_index("c")
    pltpu.sync_copy(bvec_hbm.at[0], bvbuf)
    bv = bvbuf[pl.ds(0, 16)]

    z_i = jnp.zeros((16,), jnp.int32)

    def zbody(i, carry):
        for u in range(_U):
            hist[i, pl.ds(u * 16, 16)] = z_i
        return carry
    lax.fori_loop(0, _NB2 // 128, zbody, 0)

    base = wid * _PER
    ones = jnp.ones((16,), jnp.int32)
    bufs = (buf0, buf1)
    sems = (sem0, sem1)

    def inner(buf, acc0):
        def vbody(j, acc):
            b0 = j * (16 * _U)
            vs = [buf[pl.ds(b0 + u * 16, 16)] for u in range(_U)]
            bs = [lax.bitcast_convert_type(v, jnp.int32) for v in vs]
            hs = [lax.shift_right_logical(b, 16) for b in bs]
            rows = [jnp.bitwise_and(lax.shift_right_logical(b, 7), 511)
                    for b in bs]
            cols = [jnp.bitwise_and(b, 127) for b in bs]
            mks = [h == bv for h in hs]
            for u in range(_U):
                plsc.addupdate_scatter(hist, [rows[u], cols[u]], ones,
                                       mask=mks[u])
            for u in range(_U):
                acc = acc + jnp.where(hs[u] > bv, vs[u], 0.0)
            return acc
        return lax.fori_loop(0, _CHUNK // (16 * _U), vbody, acc0)

    acc = jnp.zeros((16,), jnp.float32)
    cp = pltpu.async_copy(loss_hbm.at[pl.ds(base, _CHUNK)], buf0, sem0)
    for c in range(_NCH):
        nxt = None
        if c + 1 < _NCH:
            nxt = pltpu.async_copy(
                loss_hbm.at[pl.ds(base + (c + 1) * _CHUNK, _CHUNK)],
                bufs[(c + 1) % 2], sems[(c + 1) % 2])
        cp.wait()
        acc = inner(bufs[c % 2], acc)
        cp = nxt

    abuf[pl.ds(0, 16)] = acc
    z_f = jnp.zeros((16,), jnp.float32)
    for u in range(1, _U):
        abuf[pl.ds(u * 16, 16)] = z_f
    pltpu.sync_copy(hist, out.at[wid])
    pltpu.sync_copy(abuf, sa_out.at[wid])


def _p5_body(cnt2_ref, sa_ref, stats_ref, out_ref):
    sv = stats_ref[...]
    lane = lax.broadcasted_iota(jnp.int32, (1, 128), 1)

    def pick(k):
        return jnp.sum(jnp.where(lane == k, sv, 0.0))

    B = pick(0)
    aboveB = pick(1)
    r = pick(2)
    cgt = pick(4)
    sgt = pick(5)
    sumAbove = jnp.sum(sa_ref[...])

    h = cnt2_ref[0].astype(jnp.float32)
    for i in range(1, _NT):
        h = h + cnt2_ref[i].astype(jnp.float32)
    S = _suffix(h)
    R = _NB2 // 128
    ii = lax.broadcasted_iota(jnp.int32, (R, 128), 0)
    jj = lax.broadcasted_iota(jnp.int32, (R, 128), 1)
    flati = ii * 128 + jj
    flat = flati.astype(jnp.float32)
    L = jnp.max(jnp.where(S >= r, flat, -1.0))
    hL = jnp.sum(jnp.where(flat == L, h, 0.0))
    SL = jnp.sum(jnp.where(flat == L, S, 0.0))
    cnt_gt_t = aboveB + (SL - hL)
    bbits = lax.shift_left(B.astype(jnp.int32), 16)
    vals = lax.bitcast_convert_type(jnp.bitwise_or(bbits, flati), jnp.float32)
    t = jnp.sum(jnp.where(flat == L, vals, 0.0))
    sum_gt_t = sumAbove + jnp.sum(jnp.where(flat > L, h * vals, 0.0))
    else_ans = (sum_gt_t + (_N_MIN - cnt_gt_t) * t) / _N_MIN
    if_ans = sgt / jnp.maximum(cgt, 1.0)
    ans = jnp.where(cgt > _N_MIN, if_ans, else_ans)
    out_ref[...] = jnp.full((1, 128), ans, jnp.float32)


_p5 = pl.pallas_call(
    _p5_body,
    out_shape=jax.ShapeDtypeStruct((1, 128), jnp.float32),
)


@jax.jit
def kernel(logits, labels):
    loss, sumP, cntP = _p1(logits, labels)
    lossf = loss.reshape(_N)
    cnt1 = _hist1(lossf)
    stats, bvec = _p3(cnt1, sumP, cntP)
    cnt2, sa = _hist2(lossf, bvec)
    out = _p5(cnt2, sa, stats)
    return out[0, 0]
